# Initial kernel scaffold; baseline (speedup 1.0000x reference)
import jax
import jax.numpy as jnp
from jax import lax
from jax.experimental import pallas as pl
from jax.experimental.pallas import tpu as pltpu
from jax.experimental.pallas import tpu_sc as plsc

_N = 10000
_E = 160000
_D = 256
_H = 16
_C = 40

_NC = 2
_NS = 16
_NW = _NC * _NS
_EPW = _E // _NW
_CH = 125
_NCHUNK = _EPW // _CH
_RPS = _N // _NS


def _sc_aggregate(table, srcs, dsts, with_cnt):
    mesh = plsc.VectorSubcoreMesh(core_axis_name="c", subcore_axis_name="s")

    out_type = [jax.ShapeDtypeStruct((_NC, _N, _H), jnp.float32)]
    scratch = [
        pltpu.VMEM((_NCHUNK, _CH), jnp.int32),
        pltpu.VMEM((_NCHUNK, _CH), jnp.int32),
        pltpu.VMEM((_CH, _H), jnp.float32),
        pltpu.VMEM((_CH, _H), jnp.float32),
        pltpu.VMEM((_RPS, _H), jnp.float32),
        pltpu.VMEM_SHARED((_N, _H), jnp.float32),
        pltpu.SemaphoreType.DMA,
        pltpu.SemaphoreType.DMA,
    ]
    if with_cnt:
        out_type.append(jax.ShapeDtypeStruct((_NC, _N, _H), jnp.float32))
        scratch.append(pltpu.VMEM((_CH, _H), jnp.float32))
        scratch.append(pltpu.VMEM_SHARED((_N, _H), jnp.float32))

    def body(table_hbm, srcs_hbm, dsts_hbm, *rest):
        if with_cnt:
            (out_sum, out_cnt, idx_s, idx_d, gbuf_a, gbuf_b, zbuf,
             acc, sem_a, sem_b, ones, acc_cnt) = rest
        else:
            (out_sum, idx_s, idx_d, gbuf_a, gbuf_b, zbuf,
             acc, sem_a, sem_b) = rest

        cid = lax.axis_index("c")
        sid = lax.axis_index("s")
        wid = sid * _NC + cid

        pltpu.sync_copy(srcs_hbm.at[wid], idx_s)
        pltpu.sync_copy(dsts_hbm.at[wid], idx_d)

        def zrow(i, _):
            zbuf[i, :] = jnp.zeros((_H,), jnp.float32)
            return 0
        lax.fori_loop(0, _RPS, zrow, 0)
        if with_cnt:
            def orow(i, _):
                ones[i, :] = jnp.ones((_H,), jnp.float32)
                return 0
            lax.fori_loop(0, _CH, orow, 0)

        pltpu.sync_copy(zbuf, acc.at[pl.ds(sid * _RPS, _RPS)])
        if with_cnt:
            pltpu.sync_copy(zbuf, acc_cnt.at[pl.ds(sid * _RPS, _RPS)])
        plsc.subcore_barrier()

        pltpu.make_async_copy(table_hbm.at[idx_s.at[0]], gbuf_a, sem_a).start()

        def chunk(j, _):
            use_a = lax.rem(j, 2) == 0
            nxt = j + 1

            @pl.when(jnp.logical_and(nxt < _NCHUNK, use_a))
            def _():
                pltpu.make_async_copy(
                    table_hbm.at[idx_s.at[nxt]], gbuf_b, sem_b).start()

            @pl.when(jnp.logical_and(nxt < _NCHUNK, jnp.logical_not(use_a)))
            def _():
                pltpu.make_async_copy(
                    table_hbm.at[idx_s.at[nxt]], gbuf_a, sem_a).start()

            @pl.when(use_a)
            def _():
                pltpu.make_async_copy(
                    table_hbm.at[idx_s.at[j]], gbuf_a, sem_a).wait()
                pltpu.sync_copy(gbuf_a, acc.at[idx_d.at[j]], add=True)

            @pl.when(jnp.logical_not(use_a))
            def _():
                pltpu.make_async_copy(
                    table_hbm.at[idx_s.at[j]], gbuf_b, sem_b).wait()
                pltpu.sync_copy(gbuf_b, acc.at[idx_d.at[j]], add=True)

            if with_cnt:
                pltpu.sync_copy(ones, acc_cnt.at[idx_d.at[j]], add=True)
            return 0

        lax.fori_loop(0, _NCHUNK, chunk, 0)
        plsc.subcore_barrier()

        sl = pl.ds(sid * _RPS, _RPS)
        pltpu.sync_copy(acc.at[sl], out_sum.at[cid, sl])
        if with_cnt:
            pltpu.sync_copy(acc_cnt.at[sl], out_cnt.at[cid, sl])

    fn = pl.kernel(body, out_type=out_type, mesh=mesh,
                   scratch_types=scratch)
    return fn(table, srcs, dsts)


def _tc_project(x, wl_t, wr_t):
    bm = 1000

    def body(x_ref, wl_ref, wr_ref, y_ref, xr_ref):
        xv = x_ref[...]
        y_ref[...] = jnp.dot(xv, wl_ref[...],
                             preferred_element_type=jnp.float32)
        xr_ref[...] = jnp.dot(xv, wr_ref[...],
                              preferred_element_type=jnp.float32)

    return pl.pallas_call(
        body,
        grid=(_N // bm,),
        in_specs=[
            pl.BlockSpec((bm, _D), lambda i: (i, 0)),
            pl.BlockSpec((_D, _H), lambda i: (0, 0)),
            pl.BlockSpec((_D, _H), lambda i: (0, 0)),
        ],
        out_specs=[
            pl.BlockSpec((bm, _H), lambda i: (i, 0)),
            pl.BlockSpec((bm, _H), lambda i: (i, 0)),
        ],
        out_shape=[
            jax.ShapeDtypeStruct((_N, _H), jnp.float32),
            jax.ShapeDtypeStruct((_N, _H), jnp.float32),
        ],
    )(x, wl_t, wr_t)


def _tc_hidden(psum, pcnt, xr, b1):
    bm = 1000

    def body(ps_ref, pc_ref, xr_ref, b_ref, o_ref):
        s = ps_ref[0] + ps_ref[1]
        c = jnp.maximum(pc_ref[0] + pc_ref[1], 1.0)
        o_ref[...] = jnp.maximum(s / c + b_ref[...] + xr_ref[...], 0.0)

    return pl.pallas_call(
        body,
        grid=(_N // bm,),
        in_specs=[
            pl.BlockSpec((_NC, bm, _H), lambda i: (0, i, 0)),
            pl.BlockSpec((_NC, bm, _H), lambda i: (0, i, 0)),
            pl.BlockSpec((bm, _H), lambda i: (i, 0)),
            pl.BlockSpec((1, _H), lambda i: (0, 0)),
        ],
        out_specs=pl.BlockSpec((bm, _H), lambda i: (i, 0)),
        out_shape=jax.ShapeDtypeStruct((_N, _H), jnp.float32),
    )(psum, pcnt, xr, b1)


def _tc_output(psum2, pcnt, h, wl_t, b2, wr_t):
    bm = 1000

    def body(ps_ref, pc_ref, h_ref, wl_ref, b_ref, wr_ref, o_ref):
        mean = (ps_ref[0] + ps_ref[1]) / jnp.maximum(
            pc_ref[0] + pc_ref[1], 1.0)
        o = (jnp.dot(mean, wl_ref[...], preferred_element_type=jnp.float32)
             + b_ref[...]
             + jnp.dot(h_ref[...], wr_ref[...],
                       preferred_element_type=jnp.float32))
        m = jnp.max(o, axis=1, keepdims=True)
        lse = m + jnp.log(jnp.sum(jnp.exp(o - m), axis=1, keepdims=True))
        o_ref[...] = o - lse

    return pl.pallas_call(
        body,
        grid=(_N // bm,),
        in_specs=[
            pl.BlockSpec((_NC, bm, _H), lambda i: (0, i, 0)),
            pl.BlockSpec((_NC, bm, _H), lambda i: (0, i, 0)),
            pl.BlockSpec((bm, _H), lambda i: (i, 0)),
            pl.BlockSpec((_H, _C), lambda i: (0, 0)),
            pl.BlockSpec((1, _C), lambda i: (0, 0)),
            pl.BlockSpec((_H, _C), lambda i: (0, 0)),
        ],
        out_specs=pl.BlockSpec((bm, _C), lambda i: (i, 0)),
        out_shape=jax.ShapeDtypeStruct((_N, _C), jnp.float32),
    )(psum2, pcnt, h, wl_t, b2, wr_t)


def kernel(x, edge_index, W1l, b1, W1r, W2l, b2, W2r):
    srcs = edge_index[0].reshape(_NW, _NCHUNK, _CH)
    dsts = edge_index[1].reshape(_NW, _NCHUNK, _CH)

    y, xr = _tc_project(x, W1l.T, W1r.T)
    psum, pcnt = _sc_aggregate(y, srcs, dsts, with_cnt=True)
    h = _tc_hidden(psum, pcnt, xr, b1.reshape(1, _H))
    (psum2,) = _sc_aggregate(h, srcs, dsts, with_cnt=False)
    return _tc_output(psum2, pcnt, h, W2l.T, b2.reshape(1, _C), W2r.T)


# speedup vs baseline: 14.0258x; 14.0258x over previous
import jax
import jax.numpy as jnp
from jax import lax
from jax.experimental import pallas as pl
from jax.experimental.pallas import tpu as pltpu
from jax.experimental.pallas import tpu_sc as plsc

_N = 10000
_E = 160000
_D = 256
_H = 16
_C = 40

_NC = 2
_NS = 16
_NW = _NC * _NS
_EPW = _E // _NW
_CH = 125
_NCHUNK = _EPW // _CH
_NP = 10240
_RPS = _NP // _NS


def _sc_aggregate(table, srcs, dsts, with_cnt):
    mesh = plsc.VectorSubcoreMesh(core_axis_name="c", subcore_axis_name="s")

    out_type = [jax.ShapeDtypeStruct((_NC, _NP, _H), jnp.float32)]
    scratch = [
        pltpu.VMEM((_NCHUNK, _CH), jnp.int32),
        pltpu.VMEM((_NCHUNK, _CH), jnp.int32),
        pltpu.VMEM((_CH, _H), jnp.float32),
        pltpu.VMEM((_CH, _H), jnp.float32),
        pltpu.VMEM((_RPS, _H), jnp.float32),
        pltpu.VMEM_SHARED((_NP, _H), jnp.float32),
        pltpu.SemaphoreType.DMA,
        pltpu.SemaphoreType.DMA,
    ]
    if with_cnt:
        out_type.append(jax.ShapeDtypeStruct((_NC, _NP, _H), jnp.float32))
        scratch.append(pltpu.VMEM((_CH, _H), jnp.float32))
        scratch.append(pltpu.VMEM_SHARED((_NP, _H), jnp.float32))

    def body(table_hbm, srcs_hbm, dsts_hbm, *rest):
        if with_cnt:
            (out_sum, out_cnt, idx_s, idx_d, gbuf_a, gbuf_b, zbuf,
             acc, sem_a, sem_b, ones, acc_cnt) = rest
        else:
            (out_sum, idx_s, idx_d, gbuf_a, gbuf_b, zbuf,
             acc, sem_a, sem_b) = rest

        cid = lax.axis_index("c")
        sid = lax.axis_index("s")
        wid = sid * _NC + cid

        pltpu.sync_copy(srcs_hbm.at[wid], idx_s)
        pltpu.sync_copy(dsts_hbm.at[wid], idx_d)

        def zrow(i, _):
            zbuf[i, :] = jnp.zeros((_H,), jnp.float32)
            return 0
        lax.fori_loop(0, _RPS, zrow, 0)
        if with_cnt:
            def orow(i, _):
                ones[i, :] = jnp.ones((_H,), jnp.float32)
                return 0
            lax.fori_loop(0, _CH, orow, 0)

        pltpu.sync_copy(zbuf, acc.at[pl.ds(sid * _RPS, _RPS)])
        if with_cnt:
            pltpu.sync_copy(zbuf, acc_cnt.at[pl.ds(sid * _RPS, _RPS)])
        plsc.subcore_barrier()

        pltpu.make_async_copy(table_hbm.at[idx_s.at[0]], gbuf_a, sem_a).start()

        def chunk(j, _):
            use_a = lax.rem(j, 2) == 0
            nxt = j + 1

            @pl.when(jnp.logical_and(nxt < _NCHUNK, use_a))
            def _():
                pltpu.make_async_copy(
                    table_hbm.at[idx_s.at[nxt]], gbuf_b, sem_b).start()

            @pl.when(jnp.logical_and(nxt < _NCHUNK, jnp.logical_not(use_a)))
            def _():
                pltpu.make_async_copy(
                    table_hbm.at[idx_s.at[nxt]], gbuf_a, sem_a).start()

            @pl.when(use_a)
            def _():
                pltpu.make_async_copy(
                    table_hbm.at[idx_s.at[j]], gbuf_a, sem_a).wait()
                pltpu.sync_copy(gbuf_a, acc.at[idx_d.at[j]], add=True)

            @pl.when(jnp.logical_not(use_a))
            def _():
                pltpu.make_async_copy(
                    table_hbm.at[idx_s.at[j]], gbuf_b, sem_b).wait()
                pltpu.sync_copy(gbuf_b, acc.at[idx_d.at[j]], add=True)

            if with_cnt:
                pltpu.sync_copy(ones, acc_cnt.at[idx_d.at[j]], add=True)
            return 0

        lax.fori_loop(0, _NCHUNK, chunk, 0)
        plsc.subcore_barrier()

        sl = pl.ds(sid * _RPS, _RPS)
        pltpu.sync_copy(acc.at[sl], out_sum.at[cid, sl])
        if with_cnt:
            pltpu.sync_copy(acc_cnt.at[sl], out_cnt.at[cid, sl])

    fn = pl.kernel(body, out_type=out_type, mesh=mesh,
                   scratch_types=scratch,
                   compiler_params=pltpu.CompilerParams(
                       use_tc_tiling_on_sc=False))
    return fn(table, srcs, dsts)


def _tc_project(x, wl_t, wr_t):
    bm = 1000

    def body(x_ref, wl_ref, wr_ref, y_ref, xr_ref):
        xv = x_ref[...]
        y_ref[...] = jnp.dot(xv, wl_ref[...],
                             preferred_element_type=jnp.float32)
        xr_ref[...] = jnp.dot(xv, wr_ref[...],
                              preferred_element_type=jnp.float32)

    return pl.pallas_call(
        body,
        grid=(_N // bm,),
        in_specs=[
            pl.BlockSpec((bm, _D), lambda i: (i, 0)),
            pl.BlockSpec((_D, _H), lambda i: (0, 0)),
            pl.BlockSpec((_D, _H), lambda i: (0, 0)),
        ],
        out_specs=[
            pl.BlockSpec((bm, _H), lambda i: (i, 0)),
            pl.BlockSpec((bm, _H), lambda i: (i, 0)),
        ],
        out_shape=[
            jax.ShapeDtypeStruct((_N, _H), jnp.float32),
            jax.ShapeDtypeStruct((_N, _H), jnp.float32),
        ],
    )(x, wl_t, wr_t)


def _tc_hidden(psum, pcnt, xr, b1):
    bm = 1000

    def body(ps_ref, pc_ref, xr_ref, b_ref, o_ref):
        s = ps_ref[0] + ps_ref[1]
        c = jnp.maximum(pc_ref[0] + pc_ref[1], 1.0)
        o_ref[...] = jnp.maximum(s / c + b_ref[...] + xr_ref[...], 0.0)

    return pl.pallas_call(
        body,
        grid=(_N // bm,),
        in_specs=[
            pl.BlockSpec((_NC, bm, _H), lambda i: (0, i, 0)),
            pl.BlockSpec((_NC, bm, _H), lambda i: (0, i, 0)),
            pl.BlockSpec((bm, _H), lambda i: (i, 0)),
            pl.BlockSpec((1, _H), lambda i: (0, 0)),
        ],
        out_specs=pl.BlockSpec((bm, _H), lambda i: (i, 0)),
        out_shape=jax.ShapeDtypeStruct((_N, _H), jnp.float32),
    )(psum, pcnt, xr, b1)


def _tc_output(psum2, pcnt, h, wl_t, b2, wr_t):
    bm = 1000

    def body(ps_ref, pc_ref, h_ref, wl_ref, b_ref, wr_ref, o_ref):
        mean = (ps_ref[0] + ps_ref[1]) / jnp.maximum(
            pc_ref[0] + pc_ref[1], 1.0)
        o = (jnp.dot(mean, wl_ref[...], preferred_element_type=jnp.float32)
             + b_ref[...]
             + jnp.dot(h_ref[...], wr_ref[...],
                       preferred_element_type=jnp.float32))
        m = jnp.max(o, axis=1, keepdims=True)
        lse = m + jnp.log(jnp.sum(jnp.exp(o - m), axis=1, keepdims=True))
        o_ref[...] = o - lse

    return pl.pallas_call(
        body,
        grid=(_N // bm,),
        in_specs=[
            pl.BlockSpec((_NC, bm, _H), lambda i: (0, i, 0)),
            pl.BlockSpec((_NC, bm, _H), lambda i: (0, i, 0)),
            pl.BlockSpec((bm, _H), lambda i: (i, 0)),
            pl.BlockSpec((_H, _C), lambda i: (0, 0)),
            pl.BlockSpec((1, _C), lambda i: (0, 0)),
            pl.BlockSpec((_H, _C), lambda i: (0, 0)),
        ],
        out_specs=pl.BlockSpec((bm, _C), lambda i: (i, 0)),
        out_shape=jax.ShapeDtypeStruct((_N, _C), jnp.float32),
    )(psum2, pcnt, h, wl_t, b2, wr_t)


def kernel(x, edge_index, W1l, b1, W1r, W2l, b2, W2r):
    srcs = edge_index[0].reshape(_NW, _NCHUNK, _CH)
    dsts = edge_index[1].reshape(_NW, _NCHUNK, _CH)

    y, xr = _tc_project(x, W1l.T, W1r.T)
    psum, pcnt = _sc_aggregate(y, srcs, dsts, with_cnt=True)
    h = _tc_hidden(psum, pcnt, xr, b1.reshape(1, _H))
    (psum2,) = _sc_aggregate(h, srcs, dsts, with_cnt=False)
    return _tc_output(psum2, pcnt, h, W2l.T, b2.reshape(1, _C), W2r.T)


# speedup vs baseline: 15.0767x; 1.0749x over previous
import jax
import jax.numpy as jnp
from jax import lax
from jax.experimental import pallas as pl
from jax.experimental.pallas import tpu as pltpu
from jax.experimental.pallas import tpu_sc as plsc

_N = 10000
_E = 160000
_D = 256
_H = 16
_C = 40

_NC = 2
_NS = 16
_NW = _NC * _NS
_EPW = _E // _NW
_CH = 125
_NCHUNK = _EPW // _CH
_NP = 10240
_RPS = _NP // _NS


def _sc_aggregate(table, edges, with_cnt):
    mesh = plsc.VectorSubcoreMesh(core_axis_name="c", subcore_axis_name="s")

    out_type = [jax.ShapeDtypeStruct((_NC, _NP, _H), jnp.float32)]
    scratch = [
        pltpu.VMEM((_NCHUNK, _CH), jnp.int32),
        pltpu.VMEM((_NCHUNK, _CH), jnp.int32),
        pltpu.VMEM((_CH, _H), jnp.float32),
        pltpu.VMEM((_CH, _H), jnp.float32),
        pltpu.VMEM((_RPS, _H), jnp.float32),
        pltpu.VMEM_SHARED((_NP, _H), jnp.float32),
        pltpu.SemaphoreType.DMA,
        pltpu.SemaphoreType.DMA,
    ]
    if with_cnt:
        out_type.append(jax.ShapeDtypeStruct((_NC, _NP, _H), jnp.float32))
        scratch.append(pltpu.VMEM((_CH, _H), jnp.float32))
        scratch.append(pltpu.VMEM_SHARED((_NP, _H), jnp.float32))

    def body(table_hbm, edges_hbm, *rest):
        if with_cnt:
            (out_sum, out_cnt, idx_s, idx_d, gbuf_a, gbuf_b, zbuf,
             acc, sem_a, sem_b, ones, acc_cnt) = rest
        else:
            (out_sum, idx_s, idx_d, gbuf_a, gbuf_b, zbuf,
             acc, sem_a, sem_b) = rest

        cid = lax.axis_index("c")
        sid = lax.axis_index("s")
        wid = sid * _NC + cid

        pltpu.sync_copy(edges_hbm.at[0, wid], idx_s)
        pltpu.sync_copy(edges_hbm.at[1, wid], idx_d)

        def zrow(i, _):
            zbuf[i, :] = jnp.zeros((_H,), jnp.float32)
            return 0
        lax.fori_loop(0, _RPS, zrow, 0)
        if with_cnt:
            def orow(i, _):
                ones[i, :] = jnp.ones((_H,), jnp.float32)
                return 0
            lax.fori_loop(0, _CH, orow, 0)

        pltpu.sync_copy(zbuf, acc.at[pl.ds(sid * _RPS, _RPS)])
        if with_cnt:
            pltpu.sync_copy(zbuf, acc_cnt.at[pl.ds(sid * _RPS, _RPS)])
        plsc.subcore_barrier()

        pltpu.make_async_copy(table_hbm.at[idx_s.at[0]], gbuf_a, sem_a).start()

        def chunk(j, _):
            use_a = lax.rem(j, 2) == 0
            nxt = j + 1

            @pl.when(jnp.logical_and(nxt < _NCHUNK, use_a))
            def _():
                pltpu.make_async_copy(
                    table_hbm.at[idx_s.at[nxt]], gbuf_b, sem_b).start()

            @pl.when(jnp.logical_and(nxt < _NCHUNK, jnp.logical_not(use_a)))
            def _():
                pltpu.make_async_copy(
                    table_hbm.at[idx_s.at[nxt]], gbuf_a, sem_a).start()

            @pl.when(use_a)
            def _():
                pltpu.make_async_copy(
                    table_hbm.at[idx_s.at[j]], gbuf_a, sem_a).wait()
                pltpu.sync_copy(gbuf_a, acc.at[idx_d.at[j]], add=True)

            @pl.when(jnp.logical_not(use_a))
            def _():
                pltpu.make_async_copy(
                    table_hbm.at[idx_s.at[j]], gbuf_b, sem_b).wait()
                pltpu.sync_copy(gbuf_b, acc.at[idx_d.at[j]], add=True)

            if with_cnt:
                pltpu.sync_copy(ones, acc_cnt.at[idx_d.at[j]], add=True)
            return 0

        lax.fori_loop(0, _NCHUNK, chunk, 0)
        plsc.subcore_barrier()

        sl = pl.ds(sid * _RPS, _RPS)
        pltpu.sync_copy(acc.at[sl], out_sum.at[cid, sl])
        if with_cnt:
            pltpu.sync_copy(acc_cnt.at[sl], out_cnt.at[cid, sl])

    fn = pl.kernel(body, out_type=out_type, mesh=mesh,
                   scratch_types=scratch,
                   compiler_params=pltpu.CompilerParams(
                       use_tc_tiling_on_sc=False))
    return fn(table, edges)


def _tc_project(x, wl_t, wr_t):
    bm = 1000

    dn = (((1,), (1,)), ((), ()))

    def body(x_ref, wl_ref, wr_ref, y_ref, xr_ref):
        xv = x_ref[...]
        y_ref[...] = lax.dot_general(xv, wl_ref[...], dn,
                                     preferred_element_type=jnp.float32)
        xr_ref[...] = lax.dot_general(xv, wr_ref[...], dn,
                                      preferred_element_type=jnp.float32)

    return pl.pallas_call(
        body,
        grid=(_N // bm,),
        in_specs=[
            pl.BlockSpec((bm, _D), lambda i: (i, 0)),
            pl.BlockSpec((_H, _D), lambda i: (0, 0)),
            pl.BlockSpec((_H, _D), lambda i: (0, 0)),
        ],
        out_specs=[
            pl.BlockSpec((bm, _H), lambda i: (i, 0)),
            pl.BlockSpec((bm, _H), lambda i: (i, 0)),
        ],
        out_shape=[
            jax.ShapeDtypeStruct((_N, _H), jnp.float32),
            jax.ShapeDtypeStruct((_N, _H), jnp.float32),
        ],
    )(x, wl_t, wr_t)


def _tc_hidden(psum, pcnt, xr, b1):
    bm = 1000

    def body(ps_ref, pc_ref, xr_ref, b_ref, o_ref):
        s = ps_ref[0] + ps_ref[1]
        c = jnp.maximum(pc_ref[0] + pc_ref[1], 1.0)
        o_ref[...] = jnp.maximum(s / c + b_ref[...] + xr_ref[...], 0.0)

    return pl.pallas_call(
        body,
        grid=(_N // bm,),
        in_specs=[
            pl.BlockSpec((_NC, bm, _H), lambda i: (0, i, 0)),
            pl.BlockSpec((_NC, bm, _H), lambda i: (0, i, 0)),
            pl.BlockSpec((bm, _H), lambda i: (i, 0)),
            pl.BlockSpec((1, _H), lambda i: (0, 0)),
        ],
        out_specs=pl.BlockSpec((bm, _H), lambda i: (i, 0)),
        out_shape=jax.ShapeDtypeStruct((_N, _H), jnp.float32),
    )(psum, pcnt, xr, b1)


def _tc_output(psum2, pcnt, h, wl_t, b2, wr_t):
    bm = 1000

    dn = (((1,), (1,)), ((), ()))

    def body(ps_ref, pc_ref, h_ref, wl_ref, b_ref, wr_ref, o_ref):
        mean = (ps_ref[0] + ps_ref[1]) / jnp.maximum(
            pc_ref[0] + pc_ref[1], 1.0)
        o = (lax.dot_general(mean, wl_ref[...], dn,
                             preferred_element_type=jnp.float32)
             + b_ref[...]
             + lax.dot_general(h_ref[...], wr_ref[...], dn,
                               preferred_element_type=jnp.float32))
        m = jnp.max(o, axis=1, keepdims=True)
        lse = m + jnp.log(jnp.sum(jnp.exp(o - m), axis=1, keepdims=True))
        o_ref[...] = o - lse

    return pl.pallas_call(
        body,
        grid=(_N // bm,),
        in_specs=[
            pl.BlockSpec((_NC, bm, _H), lambda i: (0, i, 0)),
            pl.BlockSpec((_NC, bm, _H), lambda i: (0, i, 0)),
            pl.BlockSpec((bm, _H), lambda i: (i, 0)),
            pl.BlockSpec((_C, _H), lambda i: (0, 0)),
            pl.BlockSpec((1, _C), lambda i: (0, 0)),
            pl.BlockSpec((_C, _H), lambda i: (0, 0)),
        ],
        out_specs=pl.BlockSpec((bm, _C), lambda i: (i, 0)),
        out_shape=jax.ShapeDtypeStruct((_N, _C), jnp.float32),
    )(psum2, pcnt, h, wl_t, b2, wr_t)


def kernel(x, edge_index, W1l, b1, W1r, W2l, b2, W2r):
    edges = edge_index.reshape(2, _NW, _NCHUNK, _CH)

    y, xr = _tc_project(x, W1l, W1r)
    psum, pcnt = _sc_aggregate(y, edges, with_cnt=True)
    h = _tc_hidden(psum, pcnt, xr, b1.reshape(1, _H))
    (psum2,) = _sc_aggregate(h, edges, with_cnt=False)
    return _tc_output(psum2, pcnt, h, W2l, b2.reshape(1, _C), W2r)


# speedup vs baseline: 17.3327x; 1.1496x over previous
import jax
import jax.numpy as jnp
from jax import lax
from jax.experimental import pallas as pl
from jax.experimental.pallas import tpu as pltpu
from jax.experimental.pallas import tpu_sc as plsc

_N = 10000
_E = 160000
_D = 256
_H = 16
_C = 40

_NC = 2
_NS = 16
_NW = _NC * _NS
_EPW = _E // _NW
_CH = 125
_NCHUNK = _EPW // _CH
_NP = 10240
_RPS = _NP // _NS


def _sc_aggregate(table, edges, with_cnt):
    mesh = plsc.VectorSubcoreMesh(core_axis_name="c", subcore_axis_name="s")

    out_type = [jax.ShapeDtypeStruct((_NC, _NP, _H), jnp.float32)]
    scratch = [
        pltpu.VMEM((_NCHUNK, _CH), jnp.int32),
        pltpu.VMEM((_NCHUNK, _CH), jnp.int32),
        pltpu.VMEM((_CH, _H), jnp.float32),
        pltpu.VMEM((_CH, _H), jnp.float32),
        pltpu.VMEM((_RPS, _H), jnp.float32),
        pltpu.VMEM_SHARED((_NP, _H), jnp.float32),
        pltpu.SemaphoreType.DMA,
        pltpu.SemaphoreType.DMA,
    ]
    if with_cnt:
        out_type.append(jax.ShapeDtypeStruct((_NC, _NP, _H), jnp.float32))
        scratch.append(pltpu.VMEM((_CH, _H), jnp.float32))
        scratch.append(pltpu.VMEM_SHARED((_NP, _H), jnp.float32))

    def body(table_hbm, edges_hbm, *rest):
        if with_cnt:
            (out_sum, out_cnt, idx_s, idx_d, gbuf_a, gbuf_b, zbuf,
             acc, sem_a, sem_b, ones, acc_cnt) = rest
        else:
            (out_sum, idx_s, idx_d, gbuf_a, gbuf_b, zbuf,
             acc, sem_a, sem_b) = rest

        cid = lax.axis_index("c")
        sid = lax.axis_index("s")
        wid = sid * _NC + cid

        pltpu.sync_copy(edges_hbm.at[0, wid], idx_s)
        pltpu.sync_copy(edges_hbm.at[1, wid], idx_d)

        def zrow(i, _):
            zbuf[i, :] = jnp.zeros((_H,), jnp.float32)
            return 0
        lax.fori_loop(0, _RPS, zrow, 0)
        if with_cnt:
            def orow(i, _):
                ones[i, :] = jnp.ones((_H,), jnp.float32)
                return 0
            lax.fori_loop(0, _CH, orow, 0)

        pltpu.sync_copy(zbuf, acc.at[pl.ds(sid * _RPS, _RPS)])
        if with_cnt:
            pltpu.sync_copy(zbuf, acc_cnt.at[pl.ds(sid * _RPS, _RPS)])
        plsc.subcore_barrier()

        pltpu.make_async_copy(table_hbm.at[idx_s.at[0]], gbuf_a, sem_a).start()

        def chunk(j, _):
            use_a = lax.rem(j, 2) == 0
            nxt = j + 1

            @pl.when(jnp.logical_and(nxt < _NCHUNK, use_a))
            def _():
                pltpu.make_async_copy(
                    table_hbm.at[idx_s.at[nxt]], gbuf_b, sem_b).start()

            @pl.when(jnp.logical_and(nxt < _NCHUNK, jnp.logical_not(use_a)))
            def _():
                pltpu.make_async_copy(
                    table_hbm.at[idx_s.at[nxt]], gbuf_a, sem_a).start()

            @pl.when(use_a)
            def _():
                pltpu.make_async_copy(
                    table_hbm.at[idx_s.at[j]], gbuf_a, sem_a).wait()
                pltpu.sync_copy(gbuf_a, acc.at[idx_d.at[j]], add=True)

            @pl.when(jnp.logical_not(use_a))
            def _():
                pltpu.make_async_copy(
                    table_hbm.at[idx_s.at[j]], gbuf_b, sem_b).wait()
                pltpu.sync_copy(gbuf_b, acc.at[idx_d.at[j]], add=True)

            if with_cnt:
                pltpu.sync_copy(ones, acc_cnt.at[idx_d.at[j]], add=True)
            return 0

        lax.fori_loop(0, _NCHUNK, chunk, 0)
        plsc.subcore_barrier()

        sl = pl.ds(sid * _RPS, _RPS)
        pltpu.sync_copy(acc.at[sl], out_sum.at[cid, sl])
        if with_cnt:
            pltpu.sync_copy(acc_cnt.at[sl], out_cnt.at[cid, sl])

    fn = pl.kernel(body, out_type=out_type, mesh=mesh,
                   scratch_types=scratch,
                   compiler_params=pltpu.CompilerParams(
                       use_tc_tiling_on_sc=False))
    return fn(table, edges)


def _sc_layer2(psum, pcnt, xrb, edges):
    mesh = plsc.VectorSubcoreMesh(core_axis_name="c", subcore_axis_name="s")
    rpn = _N // _NS

    out_type = [
        jax.ShapeDtypeStruct((_N, _H), jnp.float32),
        jax.ShapeDtypeStruct((_NC, _NP, _H), jnp.float32),
    ]
    scratch = [
        pltpu.VMEM((_NCHUNK, _CH), jnp.int32),
        pltpu.VMEM((_NCHUNK, _CH), jnp.int32),
        pltpu.VMEM((_CH, _H), jnp.float32),
        pltpu.VMEM((_CH, _H), jnp.float32),
        pltpu.VMEM((rpn, _H), jnp.float32),
        pltpu.VMEM((rpn, _H), jnp.float32),
        pltpu.VMEM((rpn, _H), jnp.float32),
        pltpu.VMEM((rpn, _H), jnp.float32),
        pltpu.VMEM((rpn, _H), jnp.float32),
        pltpu.VMEM((_RPS, _H), jnp.float32),
        pltpu.VMEM_SHARED((_NP, _H), jnp.float32),
        pltpu.VMEM_SHARED((_NP, _H), jnp.float32),
        pltpu.SemaphoreType.DMA,
        pltpu.SemaphoreType.DMA,
    ]

    def body(psum_hbm, pcnt_hbm, xrb_hbm, edges_hbm, h_out, m2_out,
             idx_s, idx_d, gbuf_a, gbuf_b, sbuf0, sbuf1, cbuf0, cbuf1,
             xbuf, zbuf, htab, acc2, sem_a, sem_b):
        cid = lax.axis_index("c")
        sid = lax.axis_index("s")
        wid = sid * _NC + cid
        base = sid * rpn

        pltpu.sync_copy(edges_hbm.at[0, wid], idx_s)
        pltpu.sync_copy(edges_hbm.at[1, wid], idx_d)

        pltpu.sync_copy(psum_hbm.at[0, pl.ds(base, rpn)], sbuf0)
        pltpu.sync_copy(psum_hbm.at[1, pl.ds(base, rpn)], sbuf1)
        pltpu.sync_copy(pcnt_hbm.at[0, pl.ds(base, rpn)], cbuf0)
        pltpu.sync_copy(pcnt_hbm.at[1, pl.ds(base, rpn)], cbuf1)
        pltpu.sync_copy(xrb_hbm.at[pl.ds(base, rpn)], xbuf)

        def arow(i, _):
            inv = 1.0 / jnp.maximum(cbuf0[i, :] + cbuf1[i, :], 1.0)
            s = sbuf0[i, :] + sbuf1[i, :]
            xbuf[i, :] = jnp.maximum(s * inv + xbuf[i, :], 0.0)
            cbuf0[i, :] = inv
            return 0
        lax.fori_loop(0, rpn, arow, 0)

        pltpu.sync_copy(xbuf, htab.at[pl.ds(base, rpn)])

        @pl.when(cid == 0)
        def _():
            pltpu.sync_copy(xbuf, h_out.at[pl.ds(base, rpn)])

        def zrow(i, _):
            zbuf[i, :] = jnp.zeros((_H,), jnp.float32)
            return 0
        lax.fori_loop(0, _RPS, zrow, 0)
        pltpu.sync_copy(zbuf, acc2.at[pl.ds(sid * _RPS, _RPS)])
        plsc.subcore_barrier()

        pltpu.make_async_copy(htab.at[idx_s.at[0]], gbuf_a, sem_a).start()

        def chunk(j, _):
            use_a = lax.rem(j, 2) == 0
            nxt = j + 1

            @pl.when(jnp.logical_and(nxt < _NCHUNK, use_a))
            def _():
                pltpu.make_async_copy(
                    htab.at[idx_s.at[nxt]], gbuf_b, sem_b).start()

            @pl.when(jnp.logical_and(nxt < _NCHUNK, jnp.logical_not(use_a)))
            def _():
                pltpu.make_async_copy(
                    htab.at[idx_s.at[nxt]], gbuf_a, sem_a).start()

            @pl.when(use_a)
            def _():
                pltpu.make_async_copy(
                    htab.at[idx_s.at[j]], gbuf_a, sem_a).wait()
                pltpu.sync_copy(gbuf_a, acc2.at[idx_d.at[j]], add=True)

            @pl.when(jnp.logical_not(use_a))
            def _():
                pltpu.make_async_copy(
                    htab.at[idx_s.at[j]], gbuf_b, sem_b).wait()
                pltpu.sync_copy(gbuf_b, acc2.at[idx_d.at[j]], add=True)
            return 0

        lax.fori_loop(0, _NCHUNK, chunk, 0)
        plsc.subcore_barrier()

        pltpu.sync_copy(acc2.at[pl.ds(base, rpn)], sbuf0)

        def crow(i, _):
            sbuf0[i, :] = sbuf0[i, :] * cbuf0[i, :]
            return 0
        lax.fori_loop(0, rpn, crow, 0)
        pltpu.sync_copy(sbuf0, m2_out.at[cid, pl.ds(base, rpn)])

    fn = pl.kernel(body, out_type=out_type, mesh=mesh,
                   scratch_types=scratch,
                   compiler_params=pltpu.CompilerParams(
                       use_tc_tiling_on_sc=False))
    return fn(psum, pcnt, xrb, edges)


def _tc_project(x, wl, wr, b1):
    bm = 1000

    dn = (((1,), (1,)), ((), ()))

    def body(x_ref, wl_ref, wr_ref, b_ref, y_ref, xr_ref):
        xv = x_ref[...]
        y_ref[...] = lax.dot_general(xv, wl_ref[...], dn,
                                     preferred_element_type=jnp.float32)
        xr_ref[...] = lax.dot_general(xv, wr_ref[...], dn,
                                      preferred_element_type=jnp.float32
                                      ) + b_ref[...]

    return pl.pallas_call(
        body,
        grid=(_N // bm,),
        in_specs=[
            pl.BlockSpec((bm, _D), lambda i: (i, 0)),
            pl.BlockSpec((_H, _D), lambda i: (0, 0)),
            pl.BlockSpec((_H, _D), lambda i: (0, 0)),
            pl.BlockSpec((1, _H), lambda i: (0, 0)),
        ],
        out_specs=[
            pl.BlockSpec((bm, _H), lambda i: (i, 0)),
            pl.BlockSpec((bm, _H), lambda i: (i, 0)),
        ],
        out_shape=[
            jax.ShapeDtypeStruct((_N, _H), jnp.float32),
            jax.ShapeDtypeStruct((_N, _H), jnp.float32),
        ],
    )(x, wl, wr, b1)


def _tc_output(m2, h, wl, b2, wr):
    bm = 1000

    dn = (((1,), (1,)), ((), ()))

    def body(m2_ref, h_ref, wl_ref, b_ref, wr_ref, o_ref):
        mean = m2_ref[0] + m2_ref[1]
        o = (lax.dot_general(mean, wl_ref[...], dn,
                             preferred_element_type=jnp.float32)
             + b_ref[...]
             + lax.dot_general(h_ref[...], wr_ref[...], dn,
                               preferred_element_type=jnp.float32))
        m = jnp.max(o, axis=1, keepdims=True)
        lse = m + jnp.log(jnp.sum(jnp.exp(o - m), axis=1, keepdims=True))
        o_ref[...] = o - lse

    return pl.pallas_call(
        body,
        grid=(_N // bm,),
        in_specs=[
            pl.BlockSpec((_NC, bm, _H), lambda i: (0, i, 0)),
            pl.BlockSpec((bm, _H), lambda i: (i, 0)),
            pl.BlockSpec((_C, _H), lambda i: (0, 0)),
            pl.BlockSpec((1, _C), lambda i: (0, 0)),
            pl.BlockSpec((_C, _H), lambda i: (0, 0)),
        ],
        out_specs=pl.BlockSpec((bm, _C), lambda i: (i, 0)),
        out_shape=jax.ShapeDtypeStruct((_N, _C), jnp.float32),
    )(m2, h, wl, b2, wr)


def kernel(x, edge_index, W1l, b1, W1r, W2l, b2, W2r):
    edges = edge_index.reshape(2, _NW, _NCHUNK, _CH)

    y, xrb = _tc_project(x, W1l, W1r, b1.reshape(1, _H))
    psum, pcnt = _sc_aggregate(y, edges, with_cnt=True)
    h, m2 = _sc_layer2(psum, pcnt, xrb, edges)
    return _tc_output(m2, h, W2l, b2.reshape(1, _C), W2r)


# speedup vs baseline: 18.7948x; 1.0844x over previous
import jax
import jax.numpy as jnp
from jax import lax
from jax.experimental import pallas as pl
from jax.experimental.pallas import tpu as pltpu
from jax.experimental.pallas import tpu_sc as plsc

_N = 10000
_E = 160000
_D = 256
_H = 16
_C = 40

_NC = 2
_NS = 16
_NW = _NC * _NS
_EPW = _E // _NW
_CH = 125
_NCHUNK = _EPW // _CH
_NP = 10240
_RPS = _NP // _NS


_K = 4


def _agg_pipeline(table, idx_s, idx_d, gb, gs, ss, acc,
                  ones=None, acc_cnt=None, osem=None):
    pltpu.async_copy(table.at[idx_s.at[0]], gb[0], gs[0])
    pltpu.async_copy(table.at[idx_s.at[1]], gb[1], gs[1])

    def outer(j0, _):
        for k in range(_K):
            j = j0 * _K + k
            t = (k + 2) % _K
            nxt = j + 2

            @pl.when(nxt < _NCHUNK)
            def _():
                @pl.when(j >= 2)
                def _():
                    pltpu.make_async_copy(
                        gb[t], acc.at[idx_d.at[j - 2]], ss[t]).wait()
                pltpu.async_copy(table.at[idx_s.at[nxt]], gb[t], gs[t])

            pltpu.make_async_copy(table.at[idx_s.at[j]], gb[k], gs[k]).wait()
            pltpu.async_copy(gb[k], acc.at[idx_d.at[j]], ss[k], add=True)
            if ones is not None:
                pltpu.async_copy(ones, acc_cnt.at[idx_d.at[j]], osem,
                                 add=True)
        return 0

    lax.fori_loop(0, _NCHUNK // _K, outer, 0)

    for k in range(_K):
        j = _NCHUNK - _K + k
        pltpu.make_async_copy(gb[k], acc.at[idx_d.at[j]], ss[k]).wait()
    if ones is not None:
        def drain(j, _):
            pltpu.make_async_copy(ones, acc_cnt.at[idx_d.at[j]],
                                  osem).wait()
            return 0
        lax.fori_loop(0, _NCHUNK, drain, 0)


def _sc_aggregate(table, edges, with_cnt):
    mesh = plsc.VectorSubcoreMesh(core_axis_name="c", subcore_axis_name="s")

    out_type = [jax.ShapeDtypeStruct((_NC, _NP, _H), jnp.float32)]
    scratch = (
        [pltpu.VMEM((_NCHUNK, _CH), jnp.int32)] * 2
        + [pltpu.VMEM((_CH, _H), jnp.float32)] * _K
        + [pltpu.VMEM((_RPS, _H), jnp.float32)]
        + [pltpu.VMEM_SHARED((_NP, _H), jnp.float32)]
        + [pltpu.SemaphoreType.DMA] * (2 * _K)
    )
    if with_cnt:
        out_type.append(jax.ShapeDtypeStruct((_NC, _NP, _H), jnp.float32))
        scratch.append(pltpu.VMEM((_CH, _H), jnp.float32))
        scratch.append(pltpu.VMEM_SHARED((_NP, _H), jnp.float32))
        scratch.append(pltpu.SemaphoreType.DMA)

    def body(table_hbm, edges_hbm, *rest):
        if with_cnt:
            out_sum, out_cnt = rest[0], rest[1]
            rest = rest[2:]
            ones, acc_cnt, osem = rest[-3:]
        else:
            out_sum = rest[0]
            rest = rest[1:]
            ones = acc_cnt = osem = None
        idx_s, idx_d = rest[0], rest[1]
        gb = rest[2:2 + _K]
        zbuf = rest[2 + _K]
        acc = rest[3 + _K]
        gs = rest[4 + _K:4 + 2 * _K]
        ss = rest[4 + 2 * _K:4 + 3 * _K]

        cid = lax.axis_index("c")
        sid = lax.axis_index("s")
        wid = sid * _NC + cid

        pltpu.sync_copy(edges_hbm.at[0, wid], idx_s)
        pltpu.sync_copy(edges_hbm.at[1, wid], idx_d)

        def zrow(i, _):
            zbuf[i, :] = jnp.zeros((_H,), jnp.float32)
            return 0
        lax.fori_loop(0, _RPS, zrow, 0)
        if with_cnt:
            def orow(i, _):
                ones[i, :] = jnp.ones((_H,), jnp.float32)
                return 0
            lax.fori_loop(0, _CH, orow, 0)

        pltpu.sync_copy(zbuf, acc.at[pl.ds(sid * _RPS, _RPS)])
        if with_cnt:
            pltpu.sync_copy(zbuf, acc_cnt.at[pl.ds(sid * _RPS, _RPS)])
        plsc.subcore_barrier()

        _agg_pipeline(table_hbm, idx_s, idx_d, gb, gs, ss, acc,
                      ones, acc_cnt, osem)
        plsc.subcore_barrier()

        sl = pl.ds(sid * _RPS, _RPS)
        pltpu.sync_copy(acc.at[sl], out_sum.at[cid, sl])
        if with_cnt:
            pltpu.sync_copy(acc_cnt.at[sl], out_cnt.at[cid, sl])

    fn = pl.kernel(body, out_type=out_type, mesh=mesh,
                   scratch_types=scratch,
                   compiler_params=pltpu.CompilerParams(
                       use_tc_tiling_on_sc=False))
    return fn(table, edges)


def _sc_layer2(psum, pcnt, xrb, edges):
    mesh = plsc.VectorSubcoreMesh(core_axis_name="c", subcore_axis_name="s")
    rpn = _N // _NS

    out_type = [
        jax.ShapeDtypeStruct((_N, _H), jnp.float32),
        jax.ShapeDtypeStruct((_NC, _NP, _H), jnp.float32),
    ]
    scratch = (
        [pltpu.VMEM((_NCHUNK, _CH), jnp.int32)] * 2
        + [pltpu.VMEM((_CH, _H), jnp.float32)] * _K
        + [
            pltpu.VMEM((rpn, _H), jnp.float32),
            pltpu.VMEM((rpn, _H), jnp.float32),
            pltpu.VMEM((rpn, _H), jnp.float32),
            pltpu.VMEM((rpn, _H), jnp.float32),
            pltpu.VMEM((rpn, _H), jnp.float32),
            pltpu.VMEM((_RPS, _H), jnp.float32),
            pltpu.VMEM_SHARED((_NP, _H), jnp.float32),
            pltpu.VMEM_SHARED((_NP, _H), jnp.float32),
        ]
        + [pltpu.SemaphoreType.DMA] * (2 * _K)
    )

    def body(psum_hbm, pcnt_hbm, xrb_hbm, edges_hbm, h_out, m2_out,
             idx_s, idx_d, *rest):
        gb = rest[0:_K]
        (sbuf0, sbuf1, cbuf0, cbuf1, xbuf, zbuf, htab, acc2) = \
            rest[_K:_K + 8]
        gs = rest[_K + 8:2 * _K + 8]
        ss = rest[2 * _K + 8:3 * _K + 8]
        cid = lax.axis_index("c")
        sid = lax.axis_index("s")
        wid = sid * _NC + cid
        base = sid * rpn

        pltpu.sync_copy(edges_hbm.at[0, wid], idx_s)
        pltpu.sync_copy(edges_hbm.at[1, wid], idx_d)

        pltpu.sync_copy(psum_hbm.at[0, pl.ds(base, rpn)], sbuf0)
        pltpu.sync_copy(psum_hbm.at[1, pl.ds(base, rpn)], sbuf1)
        pltpu.sync_copy(pcnt_hbm.at[0, pl.ds(base, rpn)], cbuf0)
        pltpu.sync_copy(pcnt_hbm.at[1, pl.ds(base, rpn)], cbuf1)
        pltpu.sync_copy(xrb_hbm.at[pl.ds(base, rpn)], xbuf)

        def arow(i, _):
            inv = 1.0 / jnp.maximum(cbuf0[i, :] + cbuf1[i, :], 1.0)
            s = sbuf0[i, :] + sbuf1[i, :]
            xbuf[i, :] = jnp.maximum(s * inv + xbuf[i, :], 0.0)
            cbuf0[i, :] = inv
            return 0
        lax.fori_loop(0, rpn, arow, 0)

        pltpu.sync_copy(xbuf, htab.at[pl.ds(base, rpn)])

        @pl.when(cid == 0)
        def _():
            pltpu.sync_copy(xbuf, h_out.at[pl.ds(base, rpn)])

        def zrow(i, _):
            zbuf[i, :] = jnp.zeros((_H,), jnp.float32)
            return 0
        lax.fori_loop(0, _RPS, zrow, 0)
        pltpu.sync_copy(zbuf, acc2.at[pl.ds(sid * _RPS, _RPS)])
        plsc.subcore_barrier()

        _agg_pipeline(htab, idx_s, idx_d, gb, gs, ss, acc2)
        plsc.subcore_barrier()

        pltpu.sync_copy(acc2.at[pl.ds(base, rpn)], sbuf0)

        def crow(i, _):
            sbuf0[i, :] = sbuf0[i, :] * cbuf0[i, :]
            return 0
        lax.fori_loop(0, rpn, crow, 0)
        pltpu.sync_copy(sbuf0, m2_out.at[cid, pl.ds(base, rpn)])

    fn = pl.kernel(body, out_type=out_type, mesh=mesh,
                   scratch_types=scratch,
                   compiler_params=pltpu.CompilerParams(
                       use_tc_tiling_on_sc=False))
    return fn(psum, pcnt, xrb, edges)


def _tc_project(x, w, b):
    bm = 1000

    dn = (((1,), (1,)), ((), ()))

    def body(x_ref, w_ref, b_ref, o_ref):
        o_ref[...] = lax.dot_general(x_ref[...], w_ref[...], dn,
                                     preferred_element_type=jnp.float32
                                     ) + b_ref[...]

    return pl.pallas_call(
        body,
        grid=(_N // bm,),
        in_specs=[
            pl.BlockSpec((bm, _D), lambda i: (i, 0)),
            pl.BlockSpec((_H, _D), lambda i: (0, 0)),
            pl.BlockSpec((1, _H), lambda i: (0, 0)),
        ],
        out_specs=pl.BlockSpec((bm, _H), lambda i: (i, 0)),
        out_shape=jax.ShapeDtypeStruct((_N, _H), jnp.float32),
    )(x, w, b)


def _tc_output(m2, h, wl, b2, wr):
    bm = 1000

    dn = (((1,), (1,)), ((), ()))

    def body(m2_ref, h_ref, wl_ref, b_ref, wr_ref, o_ref):
        mean = m2_ref[0] + m2_ref[1]
        o = (lax.dot_general(mean, wl_ref[...], dn,
                             preferred_element_type=jnp.float32)
             + b_ref[...]
             + lax.dot_general(h_ref[...], wr_ref[...], dn,
                               preferred_element_type=jnp.float32))
        m = jnp.max(o, axis=1, keepdims=True)
        lse = m + jnp.log(jnp.sum(jnp.exp(o - m), axis=1, keepdims=True))
        o_ref[...] = o - lse

    return pl.pallas_call(
        body,
        grid=(_N // bm,),
        in_specs=[
            pl.BlockSpec((_NC, bm, _H), lambda i: (0, i, 0)),
            pl.BlockSpec((bm, _H), lambda i: (i, 0)),
            pl.BlockSpec((_C, _H), lambda i: (0, 0)),
            pl.BlockSpec((1, _C), lambda i: (0, 0)),
            pl.BlockSpec((_C, _H), lambda i: (0, 0)),
        ],
        out_specs=pl.BlockSpec((bm, _C), lambda i: (i, 0)),
        out_shape=jax.ShapeDtypeStruct((_N, _C), jnp.float32),
    )(m2, h, wl, b2, wr)


def kernel(x, edge_index, W1l, b1, W1r, W2l, b2, W2r):
    edges = edge_index.reshape(2, _NW, _NCHUNK, _CH)

    zeros_h = jnp.zeros((1, _H), jnp.float32)
    y = _tc_project(x, W1l, zeros_h)
    psum, pcnt = _sc_aggregate(y, edges, with_cnt=True)
    xrb = _tc_project(x, W1r, b1.reshape(1, _H))
    h, m2 = _sc_layer2(psum, pcnt, xrb, edges)
    return _tc_output(m2, h, W2l, b2.reshape(1, _C), W2r)


# speedup vs baseline: 20.9981x; 1.1172x over previous
import jax
import jax.numpy as jnp
from jax import lax
from jax.experimental import pallas as pl
from jax.experimental.pallas import tpu as pltpu
from jax.experimental.pallas import tpu_sc as plsc

_N = 10000
_E = 160000
_D = 256
_H = 16
_C = 40

_NC = 2
_NS = 16
_NW = _NC * _NS
_EPW = _E // _NW
_CH = 125
_NCHUNK = _EPW // _CH
_NP = 10240
_RPS = _NP // _NS


_K = 4


def _agg_pipeline(table, idx_s, idx_d, gb, gs, ss, acc,
                  ones=None, acc_cnt=None, osem=None):
    pltpu.async_copy(table.at[idx_s.at[0]], gb[0], gs[0])
    pltpu.async_copy(table.at[idx_s.at[1]], gb[1], gs[1])

    def outer(j0, _):
        for k in range(_K):
            j = j0 * _K + k
            t = (k + 2) % _K
            nxt = j + 2

            @pl.when(nxt < _NCHUNK)
            def _():
                @pl.when(j >= 2)
                def _():
                    pltpu.make_async_copy(
                        gb[t], acc.at[idx_d.at[j - 2]], ss[t]).wait()
                pltpu.async_copy(table.at[idx_s.at[nxt]], gb[t], gs[t])

            pltpu.make_async_copy(table.at[idx_s.at[j]], gb[k], gs[k]).wait()
            pltpu.async_copy(gb[k], acc.at[idx_d.at[j]], ss[k], add=True)
            if ones is not None:
                pltpu.async_copy(ones, acc_cnt.at[idx_d.at[j]], osem,
                                 add=True)
        return 0

    lax.fori_loop(0, _NCHUNK // _K, outer, 0)

    for k in range(_K):
        j = _NCHUNK - _K + k
        pltpu.make_async_copy(gb[k], acc.at[idx_d.at[j]], ss[k]).wait()
    if ones is not None:
        def drain(j, _):
            pltpu.make_async_copy(ones, acc_cnt.at[idx_d.at[j]],
                                  osem).wait()
            return 0
        lax.fori_loop(0, _NCHUNK, drain, 0)


def _sc_aggregate(table, edges, with_cnt):
    mesh = plsc.VectorSubcoreMesh(core_axis_name="c", subcore_axis_name="s")

    out_type = [jax.ShapeDtypeStruct((_NC, _NP, _H), jnp.float32)]
    scratch = (
        [pltpu.VMEM((_NCHUNK, _CH), jnp.int32)] * 2
        + [pltpu.VMEM((_CH, _H), jnp.float32)] * _K
        + [pltpu.VMEM((_RPS, _H), jnp.float32)]
        + [pltpu.VMEM_SHARED((_NP, _H), jnp.float32)]
        + [pltpu.SemaphoreType.DMA] * (2 * _K)
    )
    if with_cnt:
        out_type.append(jax.ShapeDtypeStruct((_NC, _NP, _H), jnp.float32))
        scratch.append(pltpu.VMEM((_CH, _H), jnp.float32))
        scratch.append(pltpu.VMEM_SHARED((_NP, _H), jnp.float32))
        scratch.append(pltpu.SemaphoreType.DMA)

    def body(table_hbm, edges_hbm, *rest):
        if with_cnt:
            out_sum, out_cnt = rest[0], rest[1]
            rest = rest[2:]
            ones, acc_cnt, osem = rest[-3:]
        else:
            out_sum = rest[0]
            rest = rest[1:]
            ones = acc_cnt = osem = None
        idx_s, idx_d = rest[0], rest[1]
        gb = rest[2:2 + _K]
        zbuf = rest[2 + _K]
        acc = rest[3 + _K]
        gs = rest[4 + _K:4 + 2 * _K]
        ss = rest[4 + 2 * _K:4 + 3 * _K]

        cid = lax.axis_index("c")
        sid = lax.axis_index("s")
        wid = sid * _NC + cid

        pltpu.sync_copy(edges_hbm.at[0, wid], idx_s)
        pltpu.sync_copy(edges_hbm.at[1, wid], idx_d)

        def zrow(i, _):
            zbuf[i, :] = jnp.zeros((_H,), jnp.float32)
            return 0
        lax.fori_loop(0, _RPS, zrow, 0)
        if with_cnt:
            def orow(i, _):
                ones[i, :] = jnp.ones((_H,), jnp.float32)
                return 0
            lax.fori_loop(0, _CH, orow, 0)

        pltpu.sync_copy(zbuf, acc.at[pl.ds(sid * _RPS, _RPS)])
        if with_cnt:
            pltpu.sync_copy(zbuf, acc_cnt.at[pl.ds(sid * _RPS, _RPS)])
        plsc.subcore_barrier()

        _agg_pipeline(table_hbm, idx_s, idx_d, gb, gs, ss, acc,
                      ones, acc_cnt, osem)
        plsc.subcore_barrier()

        sl = pl.ds(sid * _RPS, _RPS)
        pltpu.sync_copy(acc.at[sl], out_sum.at[cid, sl])
        if with_cnt:
            pltpu.sync_copy(acc_cnt.at[sl], out_cnt.at[cid, sl])

    fn = pl.kernel(body, out_type=out_type, mesh=mesh,
                   scratch_types=scratch,
                   compiler_params=pltpu.CompilerParams(
                       use_tc_tiling_on_sc=False))
    return fn(table, edges)


def _sc_layer2(psum, pcnt, xrb, edges):
    mesh = plsc.VectorSubcoreMesh(core_axis_name="c", subcore_axis_name="s")
    rpn = _N // _NS

    out_type = [
        jax.ShapeDtypeStruct((_N, _H), jnp.float32),
        jax.ShapeDtypeStruct((_NC, _NP, _H), jnp.float32),
    ]
    scratch = (
        [pltpu.VMEM((_NCHUNK, _CH), jnp.int32)] * 2
        + [pltpu.VMEM((_CH, _H), jnp.float32)] * _K
        + [
            pltpu.VMEM((rpn, _H), jnp.float32),
            pltpu.VMEM((rpn, _H), jnp.float32),
            pltpu.VMEM((rpn, _H), jnp.float32),
            pltpu.VMEM((rpn, _H), jnp.float32),
            pltpu.VMEM((rpn, _H), jnp.float32),
            pltpu.VMEM((_RPS, _H), jnp.float32),
            pltpu.VMEM_SHARED((_NP, _H), jnp.float32),
            pltpu.VMEM_SHARED((_NP, _H), jnp.float32),
        ]
        + [pltpu.SemaphoreType.DMA] * (2 * _K)
    )

    def body(psum_hbm, pcnt_hbm, xrb_hbm, edges_hbm, h_out, m2_out,
             idx_s, idx_d, *rest):
        gb = rest[0:_K]
        (sbuf0, sbuf1, cbuf0, cbuf1, xbuf, zbuf, htab, acc2) = \
            rest[_K:_K + 8]
        gs = rest[_K + 8:2 * _K + 8]
        ss = rest[2 * _K + 8:3 * _K + 8]
        cid = lax.axis_index("c")
        sid = lax.axis_index("s")
        wid = sid * _NC + cid
        base = sid * rpn

        pltpu.sync_copy(edges_hbm.at[0, wid], idx_s)
        pltpu.sync_copy(edges_hbm.at[1, wid], idx_d)

        pltpu.sync_copy(psum_hbm.at[0, pl.ds(base, rpn)], sbuf0)
        pltpu.sync_copy(psum_hbm.at[1, pl.ds(base, rpn)], sbuf1)
        pltpu.sync_copy(pcnt_hbm.at[0, pl.ds(base, rpn)], cbuf0)
        pltpu.sync_copy(pcnt_hbm.at[1, pl.ds(base, rpn)], cbuf1)
        pltpu.sync_copy(xrb_hbm.at[pl.ds(base, rpn)], xbuf)

        def arow(i, _):
            inv = 1.0 / jnp.maximum(cbuf0[i, :] + cbuf1[i, :], 1.0)
            s = sbuf0[i, :] + sbuf1[i, :]
            xbuf[i, :] = jnp.maximum(s * inv + xbuf[i, :], 0.0)
            cbuf0[i, :] = inv
            return 0
        lax.fori_loop(0, rpn, arow, 0)

        pltpu.sync_copy(xbuf, htab.at[pl.ds(base, rpn)])

        @pl.when(cid == 0)
        def _():
            pltpu.sync_copy(xbuf, h_out.at[pl.ds(base, rpn)])

        def zrow(i, _):
            zbuf[i, :] = jnp.zeros((_H,), jnp.float32)
            return 0
        lax.fori_loop(0, _RPS, zrow, 0)
        pltpu.sync_copy(zbuf, acc2.at[pl.ds(sid * _RPS, _RPS)])
        plsc.subcore_barrier()

        _agg_pipeline(htab, idx_s, idx_d, gb, gs, ss, acc2)
        plsc.subcore_barrier()

        pltpu.sync_copy(acc2.at[pl.ds(base, rpn)], sbuf0)

        def crow(i, _):
            sbuf0[i, :] = sbuf0[i, :] * cbuf0[i, :]
            return 0
        lax.fori_loop(0, rpn, crow, 0)
        pltpu.sync_copy(sbuf0, m2_out.at[cid, pl.ds(base, rpn)])

    fn = pl.kernel(body, out_type=out_type, mesh=mesh,
                   scratch_types=scratch,
                   compiler_params=pltpu.CompilerParams(
                       use_tc_tiling_on_sc=False))
    return fn(psum, pcnt, xrb, edges)


def _tc_project(x, w, b):
    bm = 2000
    dn = (((1,), (1,)), ((), ()))

    def body(x_ref, w_ref, b_ref, o_ref):
        o_ref[...] = lax.dot_general(x_ref[...], w_ref[...], dn,
                                     preferred_element_type=jnp.float32
                                     ) + b_ref[...]

    return pl.pallas_call(
        body,
        grid=(_N // bm,),
        in_specs=[
            pl.BlockSpec((bm, _D), lambda i: (i, 0)),
            pl.BlockSpec((_H, _D), lambda i: (0, 0)),
            pl.BlockSpec((1, _H), lambda i: (0, 0)),
        ],
        out_specs=pl.BlockSpec((bm, _H), lambda i: (i, 0)),
        out_shape=jax.ShapeDtypeStruct((_N, _H), jnp.float32),
    )(x, w, b)


def _tc_output(m2, h, wl, b2, wr):
    dn = (((1,), (1,)), ((), ()))

    def body(m2_ref, h_ref, wl_ref, b_ref, wr_ref, o_ref):
        mean = m2_ref[0, :_N, :] + m2_ref[1, :_N, :]
        o = (lax.dot_general(wl_ref[...], mean, dn,
                             preferred_element_type=jnp.float32)
             + b_ref[...]
             + lax.dot_general(wr_ref[...], h_ref[...], dn,
                               preferred_element_type=jnp.float32))
        m = jnp.max(o, axis=0, keepdims=True)
        lse = m + jnp.log(jnp.sum(jnp.exp(o - m), axis=0, keepdims=True))
        o_ref[...] = o - lse

    out_t = pl.pallas_call(
        body,
        out_shape=jax.ShapeDtypeStruct((_C, _N), jnp.float32),
    )(m2, h, wl, b2, wr)
    return out_t.T


def kernel(x, edge_index, W1l, b1, W1r, W2l, b2, W2r):
    edges = edge_index.reshape(2, _NW, _NCHUNK, _CH)

    zeros_h = jnp.zeros((1, _H), jnp.float32)
    y = _tc_project(x, W1l, zeros_h)
    psum, pcnt = _sc_aggregate(y, edges, with_cnt=True)
    xrb = _tc_project(x, W1r, b1.reshape(1, _H))
    h, m2 = _sc_layer2(psum, pcnt, xrb, edges)
    return _tc_output(m2, h, W2l, b2.reshape(_C, 1), W2r)


# speedup vs baseline: 22.0060x; 1.0480x over previous
import jax
import jax.numpy as jnp
from jax import lax
from jax.experimental import pallas as pl
from jax.experimental.pallas import tpu as pltpu
from jax.experimental.pallas import tpu_sc as plsc

_N = 10000
_E = 160000
_D = 256
_H = 16
_C = 40

_NC = 2
_NS = 16
_NW = _NC * _NS
_EPW = _E // _NW
_CH = 500
_NCHUNK = _EPW // _CH
_NP = 10240
_RPS = _NP // _NS


_K = 5


def _agg_pipeline(table, idx_s, idx_d, gb, gs, ss, acc,
                  ones=None, acc_cnt=None, osem=None):
    pltpu.async_copy(table.at[idx_s.at[0]], gb[0], gs[0])
    pltpu.async_copy(table.at[idx_s.at[1]], gb[1], gs[1])

    def outer(j0, _):
        for k in range(_K):
            j = j0 * _K + k
            t = (k + 2) % _K
            nxt = j + 2

            @pl.when(nxt < _NCHUNK)
            def _():
                @pl.when(j >= _K - 2)
                def _():
                    pltpu.make_async_copy(
                        gb[t], acc.at[idx_d.at[j - 2]], ss[t]).wait()
                pltpu.async_copy(table.at[idx_s.at[nxt]], gb[t], gs[t])

            pltpu.make_async_copy(table.at[idx_s.at[j]], gb[k], gs[k]).wait()
            pltpu.async_copy(gb[k], acc.at[idx_d.at[j]], ss[k], add=True)
            if ones is not None:
                pltpu.async_copy(ones, acc_cnt.at[idx_d.at[j]], osem,
                                 add=True)
        return 0

    lax.fori_loop(0, _NCHUNK // _K, outer, 0)

    for k in range(_K):
        j = _NCHUNK - _K + k
        pltpu.make_async_copy(gb[k], acc.at[idx_d.at[j]], ss[k]).wait()
    if ones is not None:
        def drain(j, _):
            pltpu.make_async_copy(ones, acc_cnt.at[idx_d.at[j]],
                                  osem).wait()
            return 0
        lax.fori_loop(0, _NCHUNK, drain, 0)


def _sc_aggregate(table, edges, with_cnt):
    mesh = plsc.VectorSubcoreMesh(core_axis_name="c", subcore_axis_name="s")

    out_type = [jax.ShapeDtypeStruct((_NC, _NP, _H), jnp.float32)]
    scratch = (
        [pltpu.VMEM((_NCHUNK, _CH), jnp.int32)] * 2
        + [pltpu.VMEM((_CH, _H), jnp.float32)] * _K
        + [pltpu.VMEM((_RPS, _H), jnp.float32)]
        + [pltpu.VMEM_SHARED((_NP, _H), jnp.float32)]
        + [pltpu.SemaphoreType.DMA] * (2 * _K)
    )
    if with_cnt:
        out_type.append(jax.ShapeDtypeStruct((_NC, _NP, _H), jnp.float32))
        scratch.append(pltpu.VMEM((_CH, _H), jnp.float32))
        scratch.append(pltpu.VMEM_SHARED((_NP, _H), jnp.float32))
        scratch.append(pltpu.SemaphoreType.DMA)

    def body(table_hbm, edges_hbm, *rest):
        if with_cnt:
            out_sum, out_cnt = rest[0], rest[1]
            rest = rest[2:]
            ones, acc_cnt, osem = rest[-3:]
        else:
            out_sum = rest[0]
            rest = rest[1:]
            ones = acc_cnt = osem = None
        idx_s, idx_d = rest[0], rest[1]
        gb = rest[2:2 + _K]
        zbuf = rest[2 + _K]
        acc = rest[3 + _K]
        gs = rest[4 + _K:4 + 2 * _K]
        ss = rest[4 + 2 * _K:4 + 3 * _K]

        cid = lax.axis_index("c")
        sid = lax.axis_index("s")
        wid = sid * _NC + cid

        pltpu.sync_copy(edges_hbm.at[0, wid], idx_s)
        pltpu.sync_copy(edges_hbm.at[1, wid], idx_d)

        def zrow(i, _):
            zbuf[i, :] = jnp.zeros((_H,), jnp.float32)
            return 0
        lax.fori_loop(0, _RPS, zrow, 0, unroll=8)
        if with_cnt:
            def orow(i, _):
                ones[i, :] = jnp.ones((_H,), jnp.float32)
                return 0
            lax.fori_loop(0, _CH, orow, 0, unroll=8)

        pltpu.sync_copy(zbuf, acc.at[pl.ds(sid * _RPS, _RPS)])
        if with_cnt:
            pltpu.sync_copy(zbuf, acc_cnt.at[pl.ds(sid * _RPS, _RPS)])
        plsc.subcore_barrier()

        _agg_pipeline(table_hbm, idx_s, idx_d, gb, gs, ss, acc,
                      ones, acc_cnt, osem)
        plsc.subcore_barrier()

        sl = pl.ds(sid * _RPS, _RPS)
        pltpu.sync_copy(acc.at[sl], out_sum.at[cid, sl])
        if with_cnt:
            pltpu.sync_copy(acc_cnt.at[sl], out_cnt.at[cid, sl])

    fn = pl.kernel(body, out_type=out_type, mesh=mesh,
                   scratch_types=scratch,
                   compiler_params=pltpu.CompilerParams(
                       use_tc_tiling_on_sc=False))
    return fn(table, edges)


def _sc_layer2(psum, pcnt, xrb, edges):
    mesh = plsc.VectorSubcoreMesh(core_axis_name="c", subcore_axis_name="s")
    rpn = _N // _NS

    out_type = [
        jax.ShapeDtypeStruct((_N, _H), jnp.float32),
        jax.ShapeDtypeStruct((_NC, _NP, _H), jnp.float32),
    ]
    scratch = (
        [pltpu.VMEM((_NCHUNK, _CH), jnp.int32)] * 2
        + [pltpu.VMEM((_CH, _H), jnp.float32)] * _K
        + [
            pltpu.VMEM((rpn, _H), jnp.float32),
            pltpu.VMEM((rpn, _H), jnp.float32),
            pltpu.VMEM((rpn, _H), jnp.float32),
            pltpu.VMEM((rpn, _H), jnp.float32),
            pltpu.VMEM((rpn, _H), jnp.float32),
            pltpu.VMEM((_RPS, _H), jnp.float32),
            pltpu.VMEM_SHARED((_NP, _H), jnp.float32),
            pltpu.VMEM_SHARED((_NP, _H), jnp.float32),
        ]
        + [pltpu.SemaphoreType.DMA] * (2 * _K)
    )

    def body(psum_hbm, pcnt_hbm, xrb_hbm, edges_hbm, h_out, m2_out,
             idx_s, idx_d, *rest):
        gb = rest[0:_K]
        (sbuf0, sbuf1, cbuf0, cbuf1, xbuf, zbuf, htab, acc2) = \
            rest[_K:_K + 8]
        gs = rest[_K + 8:2 * _K + 8]
        ss = rest[2 * _K + 8:3 * _K + 8]
        cid = lax.axis_index("c")
        sid = lax.axis_index("s")
        wid = sid * _NC + cid
        base = sid * rpn

        pltpu.sync_copy(edges_hbm.at[0, wid], idx_s)
        pltpu.sync_copy(edges_hbm.at[1, wid], idx_d)

        pltpu.sync_copy(psum_hbm.at[0, pl.ds(base, rpn)], sbuf0)
        pltpu.sync_copy(psum_hbm.at[1, pl.ds(base, rpn)], sbuf1)
        pltpu.sync_copy(pcnt_hbm.at[0, pl.ds(base, rpn)], cbuf0)
        pltpu.sync_copy(pcnt_hbm.at[1, pl.ds(base, rpn)], cbuf1)
        pltpu.sync_copy(xrb_hbm.at[pl.ds(base, rpn)], xbuf)

        def arow(i, _):
            inv = 1.0 / jnp.maximum(cbuf0[i, :] + cbuf1[i, :], 1.0)
            s = sbuf0[i, :] + sbuf1[i, :]
            xbuf[i, :] = jnp.maximum(s * inv + xbuf[i, :], 0.0)
            cbuf0[i, :] = inv
            return 0
        lax.fori_loop(0, rpn, arow, 0, unroll=5)

        pltpu.sync_copy(xbuf, htab.at[pl.ds(base, rpn)])

        @pl.when(cid == 0)
        def _():
            pltpu.sync_copy(xbuf, h_out.at[pl.ds(base, rpn)])

        def zrow(i, _):
            zbuf[i, :] = jnp.zeros((_H,), jnp.float32)
            return 0
        lax.fori_loop(0, _RPS, zrow, 0, unroll=8)
        pltpu.sync_copy(zbuf, acc2.at[pl.ds(sid * _RPS, _RPS)])
        plsc.subcore_barrier()

        _agg_pipeline(htab, idx_s, idx_d, gb, gs, ss, acc2)
        plsc.subcore_barrier()

        pltpu.sync_copy(acc2.at[pl.ds(base, rpn)], sbuf0)

        def crow(i, _):
            sbuf0[i, :] = sbuf0[i, :] * cbuf0[i, :]
            return 0
        lax.fori_loop(0, rpn, crow, 0, unroll=5)
        pltpu.sync_copy(sbuf0, m2_out.at[cid, pl.ds(base, rpn)])

    fn = pl.kernel(body, out_type=out_type, mesh=mesh,
                   scratch_types=scratch,
                   compiler_params=pltpu.CompilerParams(
                       use_tc_tiling_on_sc=False))
    return fn(psum, pcnt, xrb, edges)


def _tc_project(x, w, b):
    bm = 2000
    dn = (((1,), (1,)), ((), ()))

    def body(x_ref, w_ref, b_ref, o_ref):
        o_ref[...] = lax.dot_general(x_ref[...], w_ref[...], dn,
                                     preferred_element_type=jnp.float32
                                     ) + b_ref[...]

    return pl.pallas_call(
        body,
        grid=(_N // bm,),
        in_specs=[
            pl.BlockSpec((bm, _D), lambda i: (i, 0)),
            pl.BlockSpec((_H, _D), lambda i: (0, 0)),
            pl.BlockSpec((1, _H), lambda i: (0, 0)),
        ],
        out_specs=pl.BlockSpec((bm, _H), lambda i: (i, 0)),
        out_shape=jax.ShapeDtypeStruct((_N, _H), jnp.float32),
    )(x, w, b)


def _tc_output(m2, h, wl, b2, wr):
    dn = (((1,), (1,)), ((), ()))

    def body(m2_ref, h_ref, wl_ref, b_ref, wr_ref, o_ref):
        mean = m2_ref[0, :_N, :] + m2_ref[1, :_N, :]
        o = (lax.dot_general(wl_ref[...], mean, dn,
                             preferred_element_type=jnp.float32)
             + b_ref[...]
             + lax.dot_general(wr_ref[...], h_ref[...], dn,
                               preferred_element_type=jnp.float32))
        m = jnp.max(o, axis=0, keepdims=True)
        lse = m + jnp.log(jnp.sum(jnp.exp(o - m), axis=0, keepdims=True))
        o_ref[...] = o - lse

    out_t = pl.pallas_call(
        body,
        out_shape=jax.ShapeDtypeStruct((_C, _N), jnp.float32),
    )(m2, h, wl, b2, wr)
    return out_t.T


def kernel(x, edge_index, W1l, b1, W1r, W2l, b2, W2r):
    edges = edge_index.reshape(2, _NW, _NCHUNK, _CH)

    zeros_h = jnp.zeros((1, _H), jnp.float32)
    y = _tc_project(x, W1l, zeros_h)
    psum, pcnt = _sc_aggregate(y, edges, with_cnt=True)
    xrb = _tc_project(x, W1r, b1.reshape(1, _H))
    h, m2 = _sc_layer2(psum, pcnt, xrb, edges)
    return _tc_output(m2, h, W2l, b2.reshape(_C, 1), W2r)


# speedup vs baseline: 23.0849x; 1.0490x over previous
import jax
import jax.numpy as jnp
from jax import lax
from jax.experimental import pallas as pl
from jax.experimental.pallas import tpu as pltpu
from jax.experimental.pallas import tpu_sc as plsc

_N = 10000
_E = 160000
_D = 256
_H = 16
_C = 40

_NC = 2
_NS = 16
_NW = _NC * _NS
_EPW = _E // _NW
_CH = 500
_NCHUNK = _EPW // _CH
_NP = 10240
_RPS = _NP // _NS


_K = 5


def _agg_pipeline(table, idx_s, idx_d, gb, gs, ss, acc,
                  ones=None, acc_cnt=None, osem=None):
    pltpu.async_copy(table.at[idx_s.at[0]], gb[0], gs[0])
    pltpu.async_copy(table.at[idx_s.at[1]], gb[1], gs[1])

    def outer(j0, _):
        for k in range(_K):
            j = j0 * _K + k
            t = (k + 2) % _K
            nxt = j + 2

            @pl.when(nxt < _NCHUNK)
            def _():
                @pl.when(j >= _K - 2)
                def _():
                    pltpu.make_async_copy(
                        gb[t], acc.at[idx_d.at[j - 2]], ss[t]).wait()
                pltpu.async_copy(table.at[idx_s.at[nxt]], gb[t], gs[t])

            pltpu.make_async_copy(table.at[idx_s.at[j]], gb[k], gs[k]).wait()
            pltpu.async_copy(gb[k], acc.at[idx_d.at[j]], ss[k], add=True)
            if ones is not None:
                pltpu.async_copy(ones, acc_cnt.at[idx_d.at[j]], osem,
                                 add=True)
        return 0

    lax.fori_loop(0, _NCHUNK // _K, outer, 0)

    for k in range(_K):
        j = _NCHUNK - _K + k
        pltpu.make_async_copy(gb[k], acc.at[idx_d.at[j]], ss[k]).wait()
    if ones is not None:
        def drain(j, _):
            pltpu.make_async_copy(ones, acc_cnt.at[idx_d.at[j]],
                                  osem).wait()
            return 0
        lax.fori_loop(0, _NCHUNK, drain, 0)


def _sc_aggregate(table, edges, with_cnt):
    mesh = plsc.VectorSubcoreMesh(core_axis_name="c", subcore_axis_name="s")

    out_type = [jax.ShapeDtypeStruct((_NC, _NP, _H), jnp.float32)]
    scratch = (
        [pltpu.VMEM((_NCHUNK, _CH), jnp.int32)] * 2
        + [pltpu.VMEM((_CH, _H), jnp.float32)] * _K
        + [pltpu.VMEM((_RPS, _H), jnp.float32)]
        + [pltpu.VMEM_SHARED((_NP, _H), jnp.float32)]
        + [pltpu.SemaphoreType.DMA] * (2 * _K)
    )
    if with_cnt:
        out_type.append(jax.ShapeDtypeStruct((_NC, _NP, _H), jnp.float32))
        scratch.append(pltpu.VMEM((_CH, _H), jnp.float32))
        scratch.append(pltpu.VMEM_SHARED((_NP, _H), jnp.float32))
        scratch.append(pltpu.SemaphoreType.DMA)

    def body(table_hbm, edges_hbm, *rest):
        if with_cnt:
            out_sum, out_cnt = rest[0], rest[1]
            rest = rest[2:]
            ones, acc_cnt, osem = rest[-3:]
        else:
            out_sum = rest[0]
            rest = rest[1:]
            ones = acc_cnt = osem = None
        idx_s, idx_d = rest[0], rest[1]
        gb = rest[2:2 + _K]
        zbuf = rest[2 + _K]
        acc = rest[3 + _K]
        gs = rest[4 + _K:4 + 2 * _K]
        ss = rest[4 + 2 * _K:4 + 3 * _K]

        cid = lax.axis_index("c")
        sid = lax.axis_index("s")
        wid = sid * _NC + cid

        icp = [
            pltpu.make_async_copy(edges_hbm.at[0, wid], idx_s, ss[0]),
            pltpu.make_async_copy(edges_hbm.at[1, wid], idx_d, ss[1]),
        ]
        for cp in icp:
            cp.start()

        def zrow(i, _):
            zbuf[i, :] = jnp.zeros((_H,), jnp.float32)
            return 0
        lax.fori_loop(0, _RPS, zrow, 0, unroll=8)
        if with_cnt:
            def orow(i, _):
                ones[i, :] = jnp.ones((_H,), jnp.float32)
                return 0
            lax.fori_loop(0, _CH, orow, 0, unroll=8)

        pltpu.sync_copy(zbuf, acc.at[pl.ds(sid * _RPS, _RPS)])
        if with_cnt:
            pltpu.sync_copy(zbuf, acc_cnt.at[pl.ds(sid * _RPS, _RPS)])
        for cp in icp:
            cp.wait()
        plsc.subcore_barrier()

        _agg_pipeline(table_hbm, idx_s, idx_d, gb, gs, ss, acc,
                      ones, acc_cnt, osem)
        plsc.subcore_barrier()

        sl = pl.ds(sid * _RPS, _RPS)
        pltpu.sync_copy(acc.at[sl], out_sum.at[cid, sl])
        if with_cnt:
            pltpu.sync_copy(acc_cnt.at[sl], out_cnt.at[cid, sl])

    fn = pl.kernel(body, out_type=out_type, mesh=mesh,
                   scratch_types=scratch,
                   compiler_params=pltpu.CompilerParams(
                       use_tc_tiling_on_sc=False))
    return fn(table, edges)


def _sc_layer2(psum, pcnt, xrb, edges):
    mesh = plsc.VectorSubcoreMesh(core_axis_name="c", subcore_axis_name="s")
    rpn = _N // _NS

    out_type = [
        jax.ShapeDtypeStruct((_N, _H), jnp.float32),
        jax.ShapeDtypeStruct((_NC, _NP, _H), jnp.float32),
    ]
    scratch = (
        [pltpu.VMEM((_NCHUNK, _CH), jnp.int32)] * 2
        + [pltpu.VMEM((_CH, _H), jnp.float32)] * _K
        + [
            pltpu.VMEM((rpn, _H), jnp.float32),
            pltpu.VMEM((rpn, _H), jnp.float32),
            pltpu.VMEM((rpn, _H), jnp.float32),
            pltpu.VMEM((rpn, _H), jnp.float32),
            pltpu.VMEM((rpn, _H), jnp.float32),
            pltpu.VMEM((_RPS, _H), jnp.float32),
            pltpu.VMEM_SHARED((_NP, _H), jnp.float32),
            pltpu.VMEM_SHARED((_NP, _H), jnp.float32),
        ]
        + [pltpu.SemaphoreType.DMA] * (2 * _K)
    )

    def body(psum_hbm, pcnt_hbm, xrb_hbm, edges_hbm, h_out, m2_out,
             idx_s, idx_d, *rest):
        gb = rest[0:_K]
        (sbuf0, sbuf1, cbuf0, cbuf1, xbuf, zbuf, htab, acc2) = \
            rest[_K:_K + 8]
        gs = rest[_K + 8:2 * _K + 8]
        ss = rest[2 * _K + 8:3 * _K + 8]
        cid = lax.axis_index("c")
        sid = lax.axis_index("s")
        wid = sid * _NC + cid
        base = sid * rpn

        icp = [
            pltpu.make_async_copy(edges_hbm.at[0, wid], idx_s, ss[0]),
            pltpu.make_async_copy(edges_hbm.at[1, wid], idx_d, ss[1]),
        ]
        for cp in icp:
            cp.start()

        cps = [
            pltpu.make_async_copy(psum_hbm.at[0, pl.ds(base, rpn)], sbuf0,
                                  gs[0]),
            pltpu.make_async_copy(psum_hbm.at[1, pl.ds(base, rpn)], sbuf1,
                                  gs[1]),
            pltpu.make_async_copy(pcnt_hbm.at[0, pl.ds(base, rpn)], cbuf0,
                                  gs[2]),
            pltpu.make_async_copy(pcnt_hbm.at[1, pl.ds(base, rpn)], cbuf1,
                                  gs[3]),
            pltpu.make_async_copy(xrb_hbm.at[pl.ds(base, rpn)], xbuf,
                                  gs[4]),
        ]
        for cp in cps:
            cp.start()
        for cp in cps:
            cp.wait()

        def arow(i, _):
            inv = 1.0 / jnp.maximum(cbuf0[i, :] + cbuf1[i, :], 1.0)
            s = sbuf0[i, :] + sbuf1[i, :]
            xbuf[i, :] = jnp.maximum(s * inv + xbuf[i, :], 0.0)
            cbuf0[i, :] = inv
            return 0
        lax.fori_loop(0, rpn, arow, 0, unroll=5)

        pltpu.sync_copy(xbuf, htab.at[pl.ds(base, rpn)])

        @pl.when(cid == 0)
        def _():
            pltpu.sync_copy(xbuf, h_out.at[pl.ds(base, rpn)])

        def zrow(i, _):
            zbuf[i, :] = jnp.zeros((_H,), jnp.float32)
            return 0
        lax.fori_loop(0, _RPS, zrow, 0, unroll=8)
        pltpu.sync_copy(zbuf, acc2.at[pl.ds(sid * _RPS, _RPS)])
        plsc.subcore_barrier()

        for cp in icp:
            cp.wait()
        _agg_pipeline(htab, idx_s, idx_d, gb, gs, ss, acc2)
        plsc.subcore_barrier()

        pltpu.sync_copy(acc2.at[pl.ds(base, rpn)], sbuf0)

        def crow(i, _):
            sbuf0[i, :] = sbuf0[i, :] * cbuf0[i, :]
            return 0
        lax.fori_loop(0, rpn, crow, 0, unroll=5)
        pltpu.sync_copy(sbuf0, m2_out.at[cid, pl.ds(base, rpn)])

    fn = pl.kernel(body, out_type=out_type, mesh=mesh,
                   scratch_types=scratch,
                   compiler_params=pltpu.CompilerParams(
                       use_tc_tiling_on_sc=False))
    return fn(psum, pcnt, xrb, edges)


def _tc_project(x, w, b):
    bm = 2000
    dn = (((1,), (1,)), ((), ()))

    def body(x_ref, w_ref, b_ref, o_ref):
        o_ref[...] = lax.dot_general(x_ref[...], w_ref[...], dn,
                                     preferred_element_type=jnp.float32
                                     ) + b_ref[...]

    return pl.pallas_call(
        body,
        grid=(_N // bm,),
        in_specs=[
            pl.BlockSpec((bm, _D), lambda i: (i, 0)),
            pl.BlockSpec((_H, _D), lambda i: (0, 0)),
            pl.BlockSpec((1, _H), lambda i: (0, 0)),
        ],
        out_specs=pl.BlockSpec((bm, _H), lambda i: (i, 0)),
        out_shape=jax.ShapeDtypeStruct((_N, _H), jnp.float32),
    )(x, w, b)


def _tc_output(m2, h, wl, b2, wr):
    dn = (((1,), (1,)), ((), ()))

    def body(m2_ref, h_ref, wl_ref, b_ref, wr_ref, o_ref):
        mean = m2_ref[0, :_N, :] + m2_ref[1, :_N, :]
        o = (lax.dot_general(wl_ref[...], mean, dn,
                             preferred_element_type=jnp.float32)
             + b_ref[...]
             + lax.dot_general(wr_ref[...], h_ref[...], dn,
                               preferred_element_type=jnp.float32))
        m = jnp.max(o, axis=0, keepdims=True)
        lse = m + jnp.log(jnp.sum(jnp.exp(o - m), axis=0, keepdims=True))
        o_ref[...] = o - lse

    out_t = pl.pallas_call(
        body,
        out_shape=jax.ShapeDtypeStruct((_C, _N), jnp.float32),
    )(m2, h, wl, b2, wr)
    return out_t.T


def kernel(x, edge_index, W1l, b1, W1r, W2l, b2, W2r):
    edges = edge_index.reshape(2, _NW, _NCHUNK, _CH)

    zeros_h = jnp.zeros((1, _H), jnp.float32)
    y = _tc_project(x, W1l, zeros_h)
    psum, pcnt = _sc_aggregate(y, edges, with_cnt=True)
    xrb = _tc_project(x, W1r, b1.reshape(1, _H))
    h, m2 = _sc_layer2(psum, pcnt, xrb, edges)
    return _tc_output(m2, h, W2l, b2.reshape(_C, 1), W2r)
